# baseline (speedup 1.0000x reference)
import jax
import jax.numpy as jnp
from jax import lax
from jax.experimental import pallas as pl
from jax.experimental.pallas import tpu as pltpu

_N = 10000
_D = 128
_S = 3


def _viterbi(obs, log_start, log_trans, means, covars):
    log_emis = -0.5 * (((obs[:, None] - means[None, :]) ** 2) / covars[None, :]
                       + jnp.log(2.0 * jnp.pi * covars)[None, :])
    init = log_start + log_emis[0]

    def step(carry, le):
        scores = carry[:, None] + log_trans
        best = jnp.max(scores, axis=0) + le
        ptr = jnp.argmax(scores, axis=0)
        return best, ptr

    final, ptrs = lax.scan(step, init, log_emis[1:])
    last = jnp.argmax(final)

    def back(carry, ptr):
        prev = ptr[carry]
        return prev, prev

    _, ys = lax.scan(back, last, ptrs, reverse=True)
    return jnp.concatenate([ys, last[None]])


def _final_pass_kernel(recon_ref, g_ref, delta_ref, out_ref):
    out_ref[...] = recon_ref[...] * g_ref[...] + delta_ref[0, 0]


def kernel(norm_x, reconstructed_features, edge_index):
    recon = reconstructed_features
    spot_mean = jnp.mean(recon, axis=1)
    s = (spot_mean - jnp.mean(spot_mean)) / (jnp.std(spot_mean) + 1e-8)
    obs = s

    t = 0.01
    startprob = jnp.array([0.1, 0.8, 0.1], dtype=jnp.float32)
    transmat = jnp.array([[1.0 - 2 * t, t, t],
                          [t, 1.0 - 2 * t, t],
                          [t, t, 1.0 - 2 * t]], dtype=jnp.float32)
    means = jnp.quantile(obs, jnp.array([0.2, 0.5, 0.8], dtype=jnp.float32))
    covars = jnp.full((_S,), jnp.var(obs) + 1e-4, dtype=jnp.float32)

    states = _viterbi(obs, jnp.log(startprob), jnp.log(transmat), means, covars)
    states_tensor = (states + 1).astype(jnp.float32)

    row, col = edge_index[0], edge_index[1]
    sums = jax.ops.segment_sum(states_tensor[row], col, num_segments=_N)
    cnt = jax.ops.segment_sum(jnp.ones_like(row, dtype=jnp.float32), col, num_segments=_N)
    neighbor_avg = jnp.where(cnt > 0, sums / jnp.maximum(cnt, 1.0), 0.0)

    smoothed = 0.5 * states_tensor + 0.5 * neighbor_avg

    R = jnp.sum(recon, axis=1)
    P = jnp.sum(norm_x, axis=1)
    a = smoothed * P / (smoothed * R + 1e-8)

    row_min = jnp.min(recon, axis=1)
    row_max = jnp.max(recon, axis=1)
    nc_row_min = jnp.minimum(a * row_min, a * row_max)
    nc_row_max = jnp.maximum(a * row_min, a * row_max)
    mn = jnp.min(nc_row_min)
    mx = jnp.max(nc_row_max)
    mean_nc = jnp.sum(a * R) / (_N * _D)

    rmin = mn * 0.8
    rmax = mx * 1.2
    c1 = (rmax - rmin) / (mx - mn + 1e-8)
    off = rmin - mn * c1
    m2 = mean_nc * c1 + off
    g = (a * c1 / m2).astype(jnp.float32)
    delta = (off / m2).astype(jnp.float32)

    out = pl.pallas_call(
        _final_pass_kernel,
        out_shape=jax.ShapeDtypeStruct((_N, _D), jnp.float32),
    )(recon, g[:, None], delta.reshape(1, 1))

    reg_loss = jnp.sum(recon ** 2) * 1e-4
    return out, reg_loss


# speedup vs baseline: 22.5295x; 22.5295x over previous
import math

import jax
import jax.numpy as jnp
from jax import lax
from jax.experimental import pallas as pl
from jax.experimental.pallas import tpu as pltpu

_N = 10000
_D = 128
_S = 3

_TCH = 79
_CH = 128
_NP = _TCH * _CH

_T = 0.01
_LT = [[math.log(1.0 - 2 * _T) if i == j else math.log(_T) for j in range(3)]
       for i in range(3)]
_LS = [math.log(0.1), math.log(0.8), math.log(0.1)]
_BIG = -1e30


def _vit_kernel(obs_ref, sc_ref, out_ref, p0, p1, p2, w0, w1, w2):
    lane = lax.broadcasted_iota(jnp.int32, (1, _CH), 1)
    m = [sc_ref[0], sc_ref[1], sc_ref[2]]
    inv_cv = sc_ref[3]
    lcv = sc_ref[4]
    fbig = jnp.full((1, _CH), _BIG, jnp.float32)
    fzero = jnp.zeros((1, _CH), jnp.float32)

    def le_row(tl):
        o = obs_ref[pl.ds(tl, 1), :]
        return [-0.5 * ((o - m[j]) * (o - m[j]) * inv_cv + lcv) for j in range(3)]

    def valid_row(tl):
        return (lane * _TCH + tl) < _N

    def stepA(tl, M):
        le = le_row(tl)
        valid = valid_row(tl)
        is_first = jnp.logical_and(lane == 0, tl == 0)
        S = []
        for i in range(3):
            for j in range(3):
                if i == j:
                    v = jnp.where(is_first, le[j], _LT[i][j] + le[j])
                    v = jnp.where(valid, v, fzero)
                else:
                    v = jnp.where(is_first, fbig, _LT[i][j] + le[j])
                    v = jnp.where(valid, v, fbig)
                S.append(v)
        out = []
        for i in range(3):
            for j in range(3):
                out.append(jnp.maximum(
                    jnp.maximum(M[3 * i + 0] + S[0 * 3 + j],
                                M[3 * i + 1] + S[1 * 3 + j]),
                    M[3 * i + 2] + S[2 * 3 + j]))
        return tuple(out)

    Minit = tuple(fzero if i == j else fbig for i in range(3) for j in range(3))
    M = lax.fori_loop(0, _TCH, stepA, Minit)

    X = list(M)
    for k in (1, 2, 4, 8, 16, 32, 64):
        sh = []
        for i in range(3):
            for j in range(3):
                r = pltpu.roll(X[3 * i + j], k, axis=1)
                idv = fzero if i == j else fbig
                sh.append(jnp.where(lane < k, idv, r))
        newX = []
        for i in range(3):
            for j in range(3):
                newX.append(jnp.maximum(
                    jnp.maximum(sh[3 * i + 0] + X[0 * 3 + j],
                                sh[3 * i + 1] + X[1 * 3 + j]),
                    sh[3 * i + 2] + X[2 * 3 + j]))
        X = newX
    Pfx = []
    for i in range(3):
        for j in range(3):
            r = pltpu.roll(X[3 * i + j], 1, axis=1)
            idv = fzero if i == j else fbig
            Pfx.append(jnp.where(lane < 1, idv, r))
    vstart = []
    for j in range(3):
        vstart.append(jnp.maximum(
            jnp.maximum(_LS[0] + Pfx[0 * 3 + j], _LS[1] + Pfx[1 * 3 + j]),
            _LS[2] + Pfx[2 * 3 + j]))

    izero = jnp.zeros((1, _CH), jnp.int32)
    ione = jnp.full((1, _CH), 1, jnp.int32)
    itwo = jnp.full((1, _CH), 2, jnp.int32)
    iconst = [izero, ione, itwo]

    def stepC(tl, v):
        le = le_row(tl)
        valid = valid_row(tl)
        is_first = jnp.logical_and(lane == 0, tl == 0)
        newv = []
        for j in range(3):
            c0 = v[0] + _LT[0][j]
            c1 = v[1] + _LT[1][j]
            c2 = v[2] + _LT[2][j]
            b = c0
            p = izero
            u1 = c1 > b
            b = jnp.where(u1, c1, b)
            p = jnp.where(u1, ione, p)
            u2 = c2 > b
            b = jnp.where(u2, c2, b)
            p = jnp.where(u2, itwo, p)
            nv = jnp.where(is_first, v[j] + le[j], b + le[j])
            nv = jnp.where(valid, nv, v[j])
            newv.append(nv)
            pstore = jnp.where(jnp.logical_and(valid, jnp.logical_not(is_first)),
                               p, iconst[j])
            pref = (p0, p1, p2)[j]
            pref[pl.ds(tl, 1), :] = pstore
        return tuple(newv)

    vend = lax.fori_loop(0, _TCH, stepC, tuple(vstart))

    lb = vend[0]
    lp = izero
    u1 = vend[1] > lb
    lb = jnp.where(u1, vend[1], lb)
    lp = jnp.where(u1, ione, lp)
    u2 = vend[2] > lb
    lp = jnp.where(u2, itwo, lp)
    last = jnp.sum(jnp.where(lane == _CH - 1, lp, izero))

    def sel_map(r0, r1, r2, s):
        return jnp.where(s == 0, r0, jnp.where(s == 1, r1, r2))

    pr0 = []
    for j in range(3):
        pref = (p0, p1, p2)[j]
        r = pltpu.roll(pref[pl.ds(0, 1), :], _CH - 1, axis=1)
        pr0.append(jnp.where(lane == _CH - 1, iconst[j], r))

    cur = [izero, ione, itwo]
    nxt = [sel_map(pr0[0], pr0[1], pr0[2], cur[s]) for s in range(3)]
    for s in range(3):
        (w0, w1, w2)[s][pl.ds(_TCH - 1, 1), :] = nxt[s]
    cur = nxt

    def stepD(i, cur):
        tl = _TCH - 2 - i
        r0 = p0[pl.ds(tl + 1, 1), :]
        r1 = p1[pl.ds(tl + 1, 1), :]
        r2 = p2[pl.ds(tl + 1, 1), :]
        n0 = sel_map(r0, r1, r2, cur[0])
        n1 = sel_map(r0, r1, r2, cur[1])
        n2 = sel_map(r0, r1, r2, cur[2])
        w0[pl.ds(tl, 1), :] = n0
        w1[pl.ds(tl, 1), :] = n1
        w2[pl.ds(tl, 1), :] = n2
        return (n0, n1, n2)

    G = lax.fori_loop(0, _TCH - 1, stepD, tuple(cur))

    H = list(G)
    for k in (1, 2, 4, 8, 16, 32, 64):
        Y = []
        for s in range(3):
            r = pltpu.roll(H[s], _CH - k, axis=1)
            Y.append(jnp.where(lane >= _CH - k, iconst[s], r))
        H = [sel_map(H[0], H[1], H[2], Y[s]) for s in range(3)]
    E = []
    for s in range(3):
        r = pltpu.roll(H[s], _CH - 1, axis=1)
        E.append(jnp.where(lane == _CH - 1, iconst[s], r))
    inc = sel_map(E[0], E[1], E[2], jnp.full((1, _CH), 1, jnp.int32) * last)

    sts = jnp.where(inc == 0, w0[...], jnp.where(inc == 1, w1[...], w2[...]))
    out_ref[...] = (sts + 1).astype(jnp.float32)


def _viterbi_states(obs, means, var):
    inv_cv = 1.0 / (var + 1e-4)
    lcv = jnp.log(2.0 * jnp.pi * (var + 1e-4))
    scalars = jnp.stack([means[0], means[1], means[2], inv_cv, lcv,
                         jnp.float32(0), jnp.float32(0), jnp.float32(0)])
    obs_pad = jnp.concatenate([obs, jnp.zeros((_NP - _N,), jnp.float32)])
    obs2d = obs_pad.reshape(_CH, _TCH).T
    out = pl.pallas_call(
        _vit_kernel,
        out_shape=jax.ShapeDtypeStruct((_TCH, _CH), jnp.float32),
        in_specs=[pl.BlockSpec(memory_space=pltpu.VMEM),
                  pl.BlockSpec(memory_space=pltpu.SMEM)],
        scratch_shapes=[pltpu.VMEM((_TCH, _CH), jnp.int32)] * 6,
    )(obs2d, scalars)
    return out.T.reshape(-1)[:_N]


def _final_pass_kernel(recon_ref, g_ref, delta_ref, out_ref):
    out_ref[...] = recon_ref[...] * g_ref[...] + delta_ref[0, 0]


def kernel(norm_x, reconstructed_features, edge_index):
    recon = reconstructed_features
    spot_mean = jnp.mean(recon, axis=1)
    obs = (spot_mean - jnp.mean(spot_mean)) / (jnp.std(spot_mean) + 1e-8)

    means = jnp.quantile(obs, jnp.array([0.2, 0.5, 0.8], dtype=jnp.float32))
    var = jnp.var(obs)

    states_tensor = _viterbi_states(obs, means, var)

    row, col = edge_index[0], edge_index[1]
    sums = jax.ops.segment_sum(states_tensor[row], col, num_segments=_N)
    cnt = jax.ops.segment_sum(jnp.ones_like(row, dtype=jnp.float32), col, num_segments=_N)
    neighbor_avg = jnp.where(cnt > 0, sums / jnp.maximum(cnt, 1.0), 0.0)

    smoothed = 0.5 * states_tensor + 0.5 * neighbor_avg

    R = jnp.sum(recon, axis=1)
    P = jnp.sum(norm_x, axis=1)
    a = smoothed * P / (smoothed * R + 1e-8)

    row_min = jnp.min(recon, axis=1)
    row_max = jnp.max(recon, axis=1)
    nc_row_min = jnp.minimum(a * row_min, a * row_max)
    nc_row_max = jnp.maximum(a * row_min, a * row_max)
    mn = jnp.min(nc_row_min)
    mx = jnp.max(nc_row_max)
    mean_nc = jnp.sum(a * R) / (_N * _D)

    rmin = mn * 0.8
    rmax = mx * 1.2
    c1 = (rmax - rmin) / (mx - mn + 1e-8)
    off = rmin - mn * c1
    m2 = mean_nc * c1 + off
    g = (a * c1 / m2).astype(jnp.float32)
    delta = (off / m2).astype(jnp.float32)

    out = pl.pallas_call(
        _final_pass_kernel,
        out_shape=jax.ShapeDtypeStruct((_N, _D), jnp.float32),
    )(recon, g[:, None], delta.reshape(1, 1))

    reg_loss = jnp.sum(recon ** 2) * 1e-4
    return out, reg_loss


# speedup vs baseline: 520.1817x; 23.0889x over previous
import functools
import math

import jax
import jax.numpy as jnp
from jax import lax
from jax.experimental import pallas as pl
from jax.experimental.pallas import tpu as pltpu
from jax.experimental.pallas import tpu_sc as plsc

_N = 10000
_D = 128
_S = 3

_TCH = 79
_CH = 128
_NP = _TCH * _CH

_T = 0.01
_LT = [[math.log(1.0 - 2 * _T) if i == j else math.log(_T) for j in range(3)]
       for i in range(3)]
_LS = [math.log(0.1), math.log(0.8), math.log(0.1)]
_BIG = -1e30


def _vit_kernel(obs_ref, sc_ref, out_ref, p0, p1, p2, w0, w1, w2):
    lane = lax.broadcasted_iota(jnp.int32, (1, _CH), 1)
    m = [sc_ref[0], sc_ref[1], sc_ref[2]]
    inv_cv = sc_ref[3]
    lcv = sc_ref[4]
    fbig = jnp.full((1, _CH), _BIG, jnp.float32)
    fzero = jnp.zeros((1, _CH), jnp.float32)

    def le_row(tl):
        o = obs_ref[pl.ds(tl, 1), :]
        return [-0.5 * ((o - m[j]) * (o - m[j]) * inv_cv + lcv) for j in range(3)]

    def valid_row(tl):
        return (lane * _TCH + tl) < _N

    def stepA(tl, M):
        le = le_row(tl)
        valid = valid_row(tl)
        is_first = jnp.logical_and(lane == 0, tl == 0)
        S = []
        for i in range(3):
            for j in range(3):
                if i == j:
                    v = jnp.where(is_first, le[j], _LT[i][j] + le[j])
                    v = jnp.where(valid, v, fzero)
                else:
                    v = jnp.where(is_first, fbig, _LT[i][j] + le[j])
                    v = jnp.where(valid, v, fbig)
                S.append(v)
        out = []
        for i in range(3):
            for j in range(3):
                out.append(jnp.maximum(
                    jnp.maximum(M[3 * i + 0] + S[0 * 3 + j],
                                M[3 * i + 1] + S[1 * 3 + j]),
                    M[3 * i + 2] + S[2 * 3 + j]))
        return tuple(out)

    Minit = tuple(fzero if i == j else fbig for i in range(3) for j in range(3))
    M = lax.fori_loop(0, _TCH, stepA, Minit)

    X = list(M)
    for k in (1, 2, 4, 8, 16, 32, 64):
        sh = []
        for i in range(3):
            for j in range(3):
                r = pltpu.roll(X[3 * i + j], k, axis=1)
                idv = fzero if i == j else fbig
                sh.append(jnp.where(lane < k, idv, r))
        newX = []
        for i in range(3):
            for j in range(3):
                newX.append(jnp.maximum(
                    jnp.maximum(sh[3 * i + 0] + X[0 * 3 + j],
                                sh[3 * i + 1] + X[1 * 3 + j]),
                    sh[3 * i + 2] + X[2 * 3 + j]))
        X = newX
    Pfx = []
    for i in range(3):
        for j in range(3):
            r = pltpu.roll(X[3 * i + j], 1, axis=1)
            idv = fzero if i == j else fbig
            Pfx.append(jnp.where(lane < 1, idv, r))
    vstart = []
    for j in range(3):
        vstart.append(jnp.maximum(
            jnp.maximum(_LS[0] + Pfx[0 * 3 + j], _LS[1] + Pfx[1 * 3 + j]),
            _LS[2] + Pfx[2 * 3 + j]))

    izero = jnp.zeros((1, _CH), jnp.int32)
    ione = jnp.full((1, _CH), 1, jnp.int32)
    itwo = jnp.full((1, _CH), 2, jnp.int32)
    iconst = [izero, ione, itwo]

    def stepC(tl, v):
        le = le_row(tl)
        valid = valid_row(tl)
        is_first = jnp.logical_and(lane == 0, tl == 0)
        newv = []
        for j in range(3):
            c0 = v[0] + _LT[0][j]
            c1 = v[1] + _LT[1][j]
            c2 = v[2] + _LT[2][j]
            b = c0
            p = izero
            u1 = c1 > b
            b = jnp.where(u1, c1, b)
            p = jnp.where(u1, ione, p)
            u2 = c2 > b
            b = jnp.where(u2, c2, b)
            p = jnp.where(u2, itwo, p)
            nv = jnp.where(is_first, v[j] + le[j], b + le[j])
            nv = jnp.where(valid, nv, v[j])
            newv.append(nv)
            pstore = jnp.where(jnp.logical_and(valid, jnp.logical_not(is_first)),
                               p, iconst[j])
            pref = (p0, p1, p2)[j]
            pref[pl.ds(tl, 1), :] = pstore
        return tuple(newv)

    vend = lax.fori_loop(0, _TCH, stepC, tuple(vstart))

    lb = vend[0]
    lp = izero
    u1 = vend[1] > lb
    lb = jnp.where(u1, vend[1], lb)
    lp = jnp.where(u1, ione, lp)
    u2 = vend[2] > lb
    lp = jnp.where(u2, itwo, lp)
    last = jnp.sum(jnp.where(lane == _CH - 1, lp, izero))

    def sel_map(r0, r1, r2, s):
        return jnp.where(s == 0, r0, jnp.where(s == 1, r1, r2))

    pr0 = []
    for j in range(3):
        pref = (p0, p1, p2)[j]
        r = pltpu.roll(pref[pl.ds(0, 1), :], _CH - 1, axis=1)
        pr0.append(jnp.where(lane == _CH - 1, iconst[j], r))

    cur = [izero, ione, itwo]
    nxt = [sel_map(pr0[0], pr0[1], pr0[2], cur[s]) for s in range(3)]
    for s in range(3):
        (w0, w1, w2)[s][pl.ds(_TCH - 1, 1), :] = nxt[s]
    cur = nxt

    def stepD(i, cur):
        tl = _TCH - 2 - i
        r0 = p0[pl.ds(tl + 1, 1), :]
        r1 = p1[pl.ds(tl + 1, 1), :]
        r2 = p2[pl.ds(tl + 1, 1), :]
        n0 = sel_map(r0, r1, r2, cur[0])
        n1 = sel_map(r0, r1, r2, cur[1])
        n2 = sel_map(r0, r1, r2, cur[2])
        w0[pl.ds(tl, 1), :] = n0
        w1[pl.ds(tl, 1), :] = n1
        w2[pl.ds(tl, 1), :] = n2
        return (n0, n1, n2)

    G = lax.fori_loop(0, _TCH - 1, stepD, tuple(cur))

    H = list(G)
    for k in (1, 2, 4, 8, 16, 32, 64):
        Y = []
        for s in range(3):
            r = pltpu.roll(H[s], _CH - k, axis=1)
            Y.append(jnp.where(lane >= _CH - k, iconst[s], r))
        H = [sel_map(H[0], H[1], H[2], Y[s]) for s in range(3)]
    E = []
    for s in range(3):
        r = pltpu.roll(H[s], _CH - 1, axis=1)
        E.append(jnp.where(lane == _CH - 1, iconst[s], r))
    inc = sel_map(E[0], E[1], E[2], jnp.full((1, _CH), 1, jnp.int32) * last)

    sts = jnp.where(inc == 0, w0[...], jnp.where(inc == 1, w1[...], w2[...]))
    out_ref[...] = (sts + 1).astype(jnp.float32)


def _viterbi_states(obs, means, var):
    inv_cv = 1.0 / (var + 1e-4)
    lcv = jnp.log(2.0 * jnp.pi * (var + 1e-4))
    scalars = jnp.stack([means[0], means[1], means[2], inv_cv, lcv,
                         jnp.float32(0), jnp.float32(0), jnp.float32(0)])
    obs_pad = jnp.concatenate([obs, jnp.zeros((_NP - _N,), jnp.float32)])
    obs2d = obs_pad.reshape(_CH, _TCH).T
    out = pl.pallas_call(
        _vit_kernel,
        out_shape=jax.ShapeDtypeStruct((_TCH, _CH), jnp.float32),
        in_specs=[pl.BlockSpec(memory_space=pltpu.VMEM),
                  pl.BlockSpec(memory_space=pltpu.SMEM)],
        scratch_shapes=[pltpu.VMEM((_TCH, _CH), jnp.int32)] * 6,
    )(obs2d, scalars)
    return out.T.reshape(-1)[:_N]


_E = 320000
_NW = 32
_EW = _E // _NW
_NPAD = 10240
_NROWS = _NPAD // 128


def _sc_scatter_body(st_hbm, row_hbm, col_hbm, out_s, out_c,
                     row_v, col_v, st_v, acc_s, acc_c, idx_v, sh_s, sh_c):
    ci = lax.axis_index("c")
    si = lax.axis_index("s")
    wid = si * 2 + ci

    iota16 = lax.iota(jnp.int32, 16)
    for kk in range(_NROWS // 16):
        idx_v[pl.ds(kk * 16, 16)] = iota16 + kk * 16

    zf = jnp.zeros((16,), jnp.float32)

    def zbody(i, carry):
        for kk in range(8):
            acc_s[i, pl.ds(kk * 16, 16)] = zf
            acc_c[i, pl.ds(kk * 16, 16)] = zf
        return carry

    lax.fori_loop(0, _NROWS, zbody, 0)

    eoff = pl.multiple_of(wid * _EW, 8)
    pltpu.sync_copy(row_hbm.at[pl.ds(eoff, _EW)], row_v)
    pltpu.sync_copy(col_hbm.at[pl.ds(eoff, _EW)], col_v)
    pltpu.sync_copy(st_hbm, st_v)

    @pl.when(si == 0)
    def _zero_shared():
        pltpu.sync_copy(acc_s, sh_s)
        pltpu.sync_copy(acc_c, sh_c)

    plsc.subcore_barrier()

    ones = jnp.full((16,), 1.0, jnp.float32)

    def ebody(i, carry):
        off = pl.multiple_of(i * 16, 16)
        r = row_v[pl.ds(off, 16)]
        c = col_v[pl.ds(off, 16)]
        v = plsc.load_gather(st_v, [r])
        rr = lax.shift_right_logical(c, 7)
        ll = lax.bitwise_and(c, 127)
        plsc.addupdate_scatter(acc_s, [rr, ll], v)
        plsc.addupdate_scatter(acc_c, [rr, ll], ones)
        return carry

    lax.fori_loop(0, _EW // 16, ebody, 0)

    pltpu.sync_copy(acc_s, sh_s.at[idx_v], add=True)
    pltpu.sync_copy(acc_c, sh_c.at[idx_v], add=True)

    plsc.subcore_barrier()

    @pl.when(si < _NROWS // 8)
    def _copy_out():
        roff = pl.multiple_of(si * 8, 8)
        ooff = pl.multiple_of(ci * _NROWS + si * 8, 8)
        pltpu.sync_copy(sh_s.at[pl.ds(roff, 8)], out_s.at[pl.ds(ooff, 8)])
        pltpu.sync_copy(sh_c.at[pl.ds(roff, 8)], out_c.at[pl.ds(ooff, 8)])


_sc_scatter = functools.partial(
    pl.kernel,
    out_type=(jax.ShapeDtypeStruct((2 * _NROWS, 128), jnp.float32),
              jax.ShapeDtypeStruct((2 * _NROWS, 128), jnp.float32)),
    mesh=plsc.VectorSubcoreMesh(core_axis_name="c", subcore_axis_name="s"),
    compiler_params=pltpu.CompilerParams(needs_layout_passes=False),
    scratch_types=[
        pltpu.VMEM((_EW,), jnp.int32),
        pltpu.VMEM((_EW,), jnp.int32),
        pltpu.VMEM((_NPAD,), jnp.float32),
        pltpu.VMEM((_NROWS, 128), jnp.float32),
        pltpu.VMEM((_NROWS, 128), jnp.float32),
        pltpu.VMEM((_NROWS,), jnp.int32),
        pltpu.VMEM_SHARED((_NROWS, 128), jnp.float32),
        pltpu.VMEM_SHARED((_NROWS, 128), jnp.float32),
    ],
)(_sc_scatter_body)


def _scatter_mean_sc(states_tensor, row, col):
    st_pad = jnp.zeros((_NPAD,), jnp.float32).at[:_N].set(states_tensor)
    part_s, part_c = _sc_scatter(st_pad, row, col)
    sums = (part_s[:_NROWS] + part_s[_NROWS:]).reshape(-1)[:_N]
    cnt = (part_c[:_NROWS] + part_c[_NROWS:]).reshape(-1)[:_N]
    return jnp.where(cnt > 0, sums / jnp.maximum(cnt, 1.0), 0.0)


def _final_pass_kernel(recon_ref, g_ref, delta_ref, out_ref):
    out_ref[...] = recon_ref[...] * g_ref[...] + delta_ref[0, 0]


def kernel(norm_x, reconstructed_features, edge_index):
    recon = reconstructed_features
    spot_mean = jnp.mean(recon, axis=1)
    obs = (spot_mean - jnp.mean(spot_mean)) / (jnp.std(spot_mean) + 1e-8)

    means = jnp.quantile(obs, jnp.array([0.2, 0.5, 0.8], dtype=jnp.float32))
    var = jnp.var(obs)

    states_tensor = _viterbi_states(obs, means, var)

    row, col = edge_index[0], edge_index[1]
    neighbor_avg = _scatter_mean_sc(states_tensor, row, col)

    smoothed = 0.5 * states_tensor + 0.5 * neighbor_avg

    R = jnp.sum(recon, axis=1)
    P = jnp.sum(norm_x, axis=1)
    a = smoothed * P / (smoothed * R + 1e-8)

    row_min = jnp.min(recon, axis=1)
    row_max = jnp.max(recon, axis=1)
    nc_row_min = jnp.minimum(a * row_min, a * row_max)
    nc_row_max = jnp.maximum(a * row_min, a * row_max)
    mn = jnp.min(nc_row_min)
    mx = jnp.max(nc_row_max)
    mean_nc = jnp.sum(a * R) / (_N * _D)

    rmin = mn * 0.8
    rmax = mx * 1.2
    c1 = (rmax - rmin) / (mx - mn + 1e-8)
    off = rmin - mn * c1
    m2 = mean_nc * c1 + off
    g = (a * c1 / m2).astype(jnp.float32)
    delta = (off / m2).astype(jnp.float32)

    out = pl.pallas_call(
        _final_pass_kernel,
        out_shape=jax.ShapeDtypeStruct((_N, _D), jnp.float32),
    )(recon, g[:, None], delta.reshape(1, 1))

    reg_loss = jnp.sum(recon ** 2) * 1e-4
    return out, reg_loss


# speedup vs baseline: 627.2370x; 1.2058x over previous
import functools
import math

import jax
import jax.numpy as jnp
from jax import lax
from jax.experimental import pallas as pl
from jax.experimental.pallas import tpu as pltpu
from jax.experimental.pallas import tpu_sc as plsc

_N = 10000
_D = 128
_S = 3

_TCH = 79
_CH = 128
_NP = _TCH * _CH

_T = 0.01
_LT = [[math.log(1.0 - 2 * _T) if i == j else math.log(_T) for j in range(3)]
       for i in range(3)]
_LS = [math.log(0.1), math.log(0.8), math.log(0.1)]
_BIG = -1e30


_RANKS = (1999.0, 2000.0, 4999.0, 5000.0, 7999.0, 8000.0)
_FRACS = (0.2 * 9999.0 - 1999.0, 0.5 * 9999.0 - 4999.0, 0.8 * 9999.0 - 7999.0)
_SGN = -2147483648


def _vit_kernel(sm_ref, out_ref, p0, p1, p2, w0, w1, w2, obs_s):
    lane = lax.broadcasted_iota(jnp.int32, (1, _CH), 1)
    fbig = jnp.full((1, _CH), _BIG, jnp.float32)
    fzero = jnp.zeros((1, _CH), jnp.float32)

    t_iota = lax.broadcasted_iota(jnp.int32, (_TCH, _CH), 0)
    c_iota = lax.broadcasted_iota(jnp.int32, (_TCH, _CH), 1)
    valid2d = (c_iota * _TCH + t_iota) < _N
    sm = sm_ref[...]
    mean = jnp.sum(jnp.where(valid2d, sm, 0.0)) / _N
    d = sm - mean
    varsm = jnp.sum(jnp.where(valid2d, d * d, 0.0)) / _N
    stde = jnp.sqrt(varsm) + 1e-8
    obs = jnp.where(valid2d, d / stde, 0.0)
    obs_s[...] = obs
    cv = varsm / (stde * stde) + 1e-4
    inv_cv = 1.0 / cv
    lcv = jnp.log(2.0 * jnp.pi * cv)

    b = lax.bitcast_convert_type(obs, jnp.int32)
    key = lax.bitwise_xor(
        b, lax.bitwise_and(lax.shift_right_arithmetic(b, 31),
                           jnp.int32(0x7FFFFFFF)))
    key = jnp.where(valid2d, key, jnp.int32(0x7FFFFFFF))

    def bs_step(i, Ps):
        bitv = lax.shift_left(jnp.int32(1), 31 - i)
        out = []
        for idx in range(6):
            C = lax.bitwise_or(Ps[idx], bitv)
            th = lax.bitwise_xor(C, jnp.int32(_SGN))
            cnt = jnp.sum(jnp.where(key < th, 1.0, 0.0))
            out.append(jnp.where(cnt <= _RANKS[idx], C, Ps[idx]))
        return tuple(out)

    Ps = lax.fori_loop(0, 32, bs_step, tuple(jnp.int32(0) for _ in range(6)))
    vs = []
    for idx in range(6):
        ks = lax.bitwise_xor(Ps[idx], jnp.int32(_SGN))
        bb = jnp.where(ks >= 0, ks, lax.bitwise_xor(ks, jnp.int32(0x7FFFFFFF)))
        vs.append(lax.bitcast_convert_type(bb, jnp.float32))
    m = [vs[0] * (1.0 - _FRACS[0]) + vs[1] * _FRACS[0],
         vs[2] * (1.0 - _FRACS[1]) + vs[3] * _FRACS[1],
         vs[4] * (1.0 - _FRACS[2]) + vs[5] * _FRACS[2]]

    def le_row(tl):
        o = obs_s[pl.ds(tl, 1), :]
        return [-0.5 * ((o - m[j]) * (o - m[j]) * inv_cv + lcv) for j in range(3)]

    def valid_row(tl):
        return (lane * _TCH + tl) < _N

    def stepA(tl, M):
        le = le_row(tl)
        valid = valid_row(tl)
        is_first = jnp.logical_and(lane == 0, tl == 0)
        S = []
        for i in range(3):
            for j in range(3):
                if i == j:
                    v = jnp.where(is_first, le[j], _LT[i][j] + le[j])
                    v = jnp.where(valid, v, fzero)
                else:
                    v = jnp.where(is_first, fbig, _LT[i][j] + le[j])
                    v = jnp.where(valid, v, fbig)
                S.append(v)
        out = []
        for i in range(3):
            for j in range(3):
                out.append(jnp.maximum(
                    jnp.maximum(M[3 * i + 0] + S[0 * 3 + j],
                                M[3 * i + 1] + S[1 * 3 + j]),
                    M[3 * i + 2] + S[2 * 3 + j]))
        return tuple(out)

    Minit = tuple(fzero if i == j else fbig for i in range(3) for j in range(3))
    M = lax.fori_loop(0, _TCH, stepA, Minit)

    X = list(M)
    for k in (1, 2, 4, 8, 16, 32, 64):
        sh = []
        for i in range(3):
            for j in range(3):
                r = pltpu.roll(X[3 * i + j], k, axis=1)
                idv = fzero if i == j else fbig
                sh.append(jnp.where(lane < k, idv, r))
        newX = []
        for i in range(3):
            for j in range(3):
                newX.append(jnp.maximum(
                    jnp.maximum(sh[3 * i + 0] + X[0 * 3 + j],
                                sh[3 * i + 1] + X[1 * 3 + j]),
                    sh[3 * i + 2] + X[2 * 3 + j]))
        X = newX
    Pfx = []
    for i in range(3):
        for j in range(3):
            r = pltpu.roll(X[3 * i + j], 1, axis=1)
            idv = fzero if i == j else fbig
            Pfx.append(jnp.where(lane < 1, idv, r))
    vstart = []
    for j in range(3):
        vstart.append(jnp.maximum(
            jnp.maximum(_LS[0] + Pfx[0 * 3 + j], _LS[1] + Pfx[1 * 3 + j]),
            _LS[2] + Pfx[2 * 3 + j]))

    izero = jnp.zeros((1, _CH), jnp.int32)
    ione = jnp.full((1, _CH), 1, jnp.int32)
    itwo = jnp.full((1, _CH), 2, jnp.int32)
    iconst = [izero, ione, itwo]

    def stepC(tl, v):
        le = le_row(tl)
        valid = valid_row(tl)
        is_first = jnp.logical_and(lane == 0, tl == 0)
        newv = []
        for j in range(3):
            c0 = v[0] + _LT[0][j]
            c1 = v[1] + _LT[1][j]
            c2 = v[2] + _LT[2][j]
            b = c0
            p = izero
            u1 = c1 > b
            b = jnp.where(u1, c1, b)
            p = jnp.where(u1, ione, p)
            u2 = c2 > b
            b = jnp.where(u2, c2, b)
            p = jnp.where(u2, itwo, p)
            nv = jnp.where(is_first, v[j] + le[j], b + le[j])
            nv = jnp.where(valid, nv, v[j])
            newv.append(nv)
            pstore = jnp.where(jnp.logical_and(valid, jnp.logical_not(is_first)),
                               p, iconst[j])
            pref = (p0, p1, p2)[j]
            pref[pl.ds(tl, 1), :] = pstore
        return tuple(newv)

    vend = lax.fori_loop(0, _TCH, stepC, tuple(vstart))

    lb = vend[0]
    lp = izero
    u1 = vend[1] > lb
    lb = jnp.where(u1, vend[1], lb)
    lp = jnp.where(u1, ione, lp)
    u2 = vend[2] > lb
    lp = jnp.where(u2, itwo, lp)
    last = jnp.sum(jnp.where(lane == _CH - 1, lp, izero))

    def sel_map(r0, r1, r2, s):
        return jnp.where(s == 0, r0, jnp.where(s == 1, r1, r2))

    pr0 = []
    for j in range(3):
        pref = (p0, p1, p2)[j]
        r = pltpu.roll(pref[pl.ds(0, 1), :], _CH - 1, axis=1)
        pr0.append(jnp.where(lane == _CH - 1, iconst[j], r))

    cur = [izero, ione, itwo]
    nxt = [sel_map(pr0[0], pr0[1], pr0[2], cur[s]) for s in range(3)]
    for s in range(3):
        (w0, w1, w2)[s][pl.ds(_TCH - 1, 1), :] = nxt[s]
    cur = nxt

    def stepD(i, cur):
        tl = _TCH - 2 - i
        r0 = p0[pl.ds(tl + 1, 1), :]
        r1 = p1[pl.ds(tl + 1, 1), :]
        r2 = p2[pl.ds(tl + 1, 1), :]
        n0 = sel_map(r0, r1, r2, cur[0])
        n1 = sel_map(r0, r1, r2, cur[1])
        n2 = sel_map(r0, r1, r2, cur[2])
        w0[pl.ds(tl, 1), :] = n0
        w1[pl.ds(tl, 1), :] = n1
        w2[pl.ds(tl, 1), :] = n2
        return (n0, n1, n2)

    G = lax.fori_loop(0, _TCH - 1, stepD, tuple(cur))

    H = list(G)
    for k in (1, 2, 4, 8, 16, 32, 64):
        Y = []
        for s in range(3):
            r = pltpu.roll(H[s], _CH - k, axis=1)
            Y.append(jnp.where(lane >= _CH - k, iconst[s], r))
        H = [sel_map(H[0], H[1], H[2], Y[s]) for s in range(3)]
    E = []
    for s in range(3):
        r = pltpu.roll(H[s], _CH - 1, axis=1)
        E.append(jnp.where(lane == _CH - 1, iconst[s], r))
    inc = sel_map(E[0], E[1], E[2], jnp.full((1, _CH), 1, jnp.int32) * last)

    sts = jnp.where(inc == 0, w0[...], jnp.where(inc == 1, w1[...], w2[...]))
    out_ref[...] = (sts + 1).astype(jnp.float32)


def _viterbi_states(sm):
    sm_pad = jnp.concatenate([sm, jnp.zeros((_NP - _N,), jnp.float32)])
    sm2d = sm_pad.reshape(_CH, _TCH).T
    out = pl.pallas_call(
        _vit_kernel,
        out_shape=jax.ShapeDtypeStruct((_TCH, _CH), jnp.float32),
        in_specs=[pl.BlockSpec(memory_space=pltpu.VMEM)],
        scratch_shapes=[pltpu.VMEM((_TCH, _CH), jnp.int32)] * 6
        + [pltpu.VMEM((_TCH, _CH), jnp.float32)],
    )(sm2d)
    return out.T.reshape(-1)[:_N]


_E = 320000
_NW = 32
_EW = _E // _NW
_NPAD = 10240
_NROWS = _NPAD // 128


def _sc_scatter_body(st_hbm, row_hbm, col_hbm, out_s, out_c,
                     row_v, col_v, st_v, acc_s, acc_c, idx_v, sh_s, sh_c):
    ci = lax.axis_index("c")
    si = lax.axis_index("s")
    wid = si * 2 + ci

    iota16 = lax.iota(jnp.int32, 16)
    for kk in range(_NROWS // 16):
        idx_v[pl.ds(kk * 16, 16)] = iota16 + kk * 16

    zf = jnp.zeros((16,), jnp.float32)

    def zbody(i, carry):
        for kk in range(8):
            acc_s[i, pl.ds(kk * 16, 16)] = zf
            acc_c[i, pl.ds(kk * 16, 16)] = zf
        return carry

    lax.fori_loop(0, _NROWS, zbody, 0)

    eoff = pl.multiple_of(wid * _EW, 8)
    pltpu.sync_copy(row_hbm.at[pl.ds(eoff, _EW)], row_v)
    pltpu.sync_copy(col_hbm.at[pl.ds(eoff, _EW)], col_v)
    pltpu.sync_copy(st_hbm, st_v)

    @pl.when(si == 0)
    def _zero_shared():
        pltpu.sync_copy(acc_s, sh_s)
        pltpu.sync_copy(acc_c, sh_c)

    plsc.subcore_barrier()

    ones = jnp.full((16,), 1.0, jnp.float32)

    def ebody(i, carry):
        off = pl.multiple_of(i * 16, 16)
        r = row_v[pl.ds(off, 16)]
        c = col_v[pl.ds(off, 16)]
        v = plsc.load_gather(st_v, [r])
        rr = lax.shift_right_logical(c, 7)
        ll = lax.bitwise_and(c, 127)
        plsc.addupdate_scatter(acc_s, [rr, ll], v)
        plsc.addupdate_scatter(acc_c, [rr, ll], ones)
        return carry

    lax.fori_loop(0, _EW // 16, ebody, 0)

    pltpu.sync_copy(acc_s, sh_s.at[idx_v], add=True)
    pltpu.sync_copy(acc_c, sh_c.at[idx_v], add=True)

    plsc.subcore_barrier()

    @pl.when(si < _NROWS // 8)
    def _copy_out():
        roff = pl.multiple_of(si * 8, 8)
        ooff = pl.multiple_of(ci * _NROWS + si * 8, 8)
        pltpu.sync_copy(sh_s.at[pl.ds(roff, 8)], out_s.at[pl.ds(ooff, 8)])
        pltpu.sync_copy(sh_c.at[pl.ds(roff, 8)], out_c.at[pl.ds(ooff, 8)])


_sc_scatter = functools.partial(
    pl.kernel,
    out_type=(jax.ShapeDtypeStruct((2 * _NROWS, 128), jnp.float32),
              jax.ShapeDtypeStruct((2 * _NROWS, 128), jnp.float32)),
    mesh=plsc.VectorSubcoreMesh(core_axis_name="c", subcore_axis_name="s"),
    compiler_params=pltpu.CompilerParams(needs_layout_passes=False),
    scratch_types=[
        pltpu.VMEM((_EW,), jnp.int32),
        pltpu.VMEM((_EW,), jnp.int32),
        pltpu.VMEM((_NPAD,), jnp.float32),
        pltpu.VMEM((_NROWS, 128), jnp.float32),
        pltpu.VMEM((_NROWS, 128), jnp.float32),
        pltpu.VMEM((_NROWS,), jnp.int32),
        pltpu.VMEM_SHARED((_NROWS, 128), jnp.float32),
        pltpu.VMEM_SHARED((_NROWS, 128), jnp.float32),
    ],
)(_sc_scatter_body)


def _scatter_mean_sc(states_tensor, row, col):
    st_pad = jnp.zeros((_NPAD,), jnp.float32).at[:_N].set(states_tensor)
    part_s, part_c = _sc_scatter(st_pad, row, col)
    sums = (part_s[:_NROWS] + part_s[_NROWS:]).reshape(-1)[:_N]
    cnt = (part_c[:_NROWS] + part_c[_NROWS:]).reshape(-1)[:_N]
    return jnp.where(cnt > 0, sums / jnp.maximum(cnt, 1.0), 0.0)


def _tail_kernel(recon_ref, nx_ref, s_ref, out_ref, reg_ref):
    recon = recon_ref[...]
    s = s_ref[...]

    R = jnp.sum(recon, axis=1, keepdims=True)
    P = jnp.sum(nx_ref[...], axis=1, keepdims=True)
    a = s * P / (s * R + 1e-8)

    row_min = jnp.min(recon, axis=1, keepdims=True)
    row_max = jnp.max(recon, axis=1, keepdims=True)
    mn = jnp.min(jnp.minimum(a * row_min, a * row_max))
    mx = jnp.max(jnp.maximum(a * row_min, a * row_max))
    mean_nc = jnp.sum(a * R) / (_N * _D)

    rmin = mn * 0.8
    rmax = mx * 1.2
    c1 = (rmax - rmin) / (mx - mn + 1e-8)
    off = rmin - mn * c1
    m2 = mean_nc * c1 + off

    out_ref[...] = recon * (a * (c1 / m2)) + off / m2
    reg_ref[...] = (jnp.sum(recon * recon) * 1e-4).reshape(1, 1)


def kernel(norm_x, reconstructed_features, edge_index):
    recon = reconstructed_features
    spot_mean = jnp.mean(recon, axis=1)

    states_tensor = _viterbi_states(spot_mean)

    row, col = edge_index[0], edge_index[1]
    neighbor_avg = _scatter_mean_sc(states_tensor, row, col)

    smoothed = 0.5 * states_tensor + 0.5 * neighbor_avg

    out, reg = pl.pallas_call(
        _tail_kernel,
        out_shape=(jax.ShapeDtypeStruct((_N, _D), jnp.float32),
                   jax.ShapeDtypeStruct((1, 1), jnp.float32)),
    )(recon, norm_x, smoothed[:, None])

    return out, reg[0, 0]


# speedup vs baseline: 655.0448x; 1.0443x over previous
import functools
import math

import jax
import jax.numpy as jnp
from jax import lax
from jax.experimental import pallas as pl
from jax.experimental.pallas import tpu as pltpu
from jax.experimental.pallas import tpu_sc as plsc

_N = 10000
_D = 128
_S = 3

_TCH = 79
_CH = 128
_NP = _TCH * _CH

_T = 0.01
_LT = [[math.log(1.0 - 2 * _T) if i == j else math.log(_T) for j in range(3)]
       for i in range(3)]
_LS = [math.log(0.1), math.log(0.8), math.log(0.1)]
_BIG = -1e30


_RANKS = (1999.0, 2000.0, 4999.0, 5000.0, 7999.0, 8000.0)
_FRACS = (0.2 * 9999.0 - 1999.0, 0.5 * 9999.0 - 4999.0, 0.8 * 9999.0 - 7999.0)
_SGN = -2147483648


def _vit_kernel(sm_ref, out_ref, p0, p1, p2, w0, w1, w2, obs_s):
    lane = lax.broadcasted_iota(jnp.int32, (1, _CH), 1)
    fbig = jnp.full((1, _CH), _BIG, jnp.float32)
    fzero = jnp.zeros((1, _CH), jnp.float32)

    t_iota = lax.broadcasted_iota(jnp.int32, (_TCH, _CH), 0)
    c_iota = lax.broadcasted_iota(jnp.int32, (_TCH, _CH), 1)
    valid2d = (c_iota * _TCH + t_iota) < _N
    sm = sm_ref[...]
    mean = jnp.sum(jnp.where(valid2d, sm, 0.0)) / _N
    d = sm - mean
    varsm = jnp.sum(jnp.where(valid2d, d * d, 0.0)) / _N
    stde = jnp.sqrt(varsm) + 1e-8
    obs = jnp.where(valid2d, d / stde, 0.0)
    obs_s[...] = obs
    cv = varsm / (stde * stde) + 1e-4
    inv_cv = 1.0 / cv
    lcv = jnp.log(2.0 * jnp.pi * cv)

    b = lax.bitcast_convert_type(obs, jnp.int32)
    key = lax.bitwise_xor(
        b, lax.bitwise_and(lax.shift_right_arithmetic(b, 31),
                           jnp.int32(0x7FFFFFFF)))
    key = jnp.where(valid2d, key, jnp.int32(0x7FFFFFFF))

    def bs_step(i, Ps):
        bitv = lax.shift_left(jnp.int32(1), 31 - i)
        out = []
        for idx in range(6):
            C = lax.bitwise_or(Ps[idx], bitv)
            th = lax.bitwise_xor(C, jnp.int32(_SGN))
            cnt = jnp.sum(jnp.where(key < th, 1.0, 0.0))
            out.append(jnp.where(cnt <= _RANKS[idx], C, Ps[idx]))
        return tuple(out)

    Ps = lax.fori_loop(0, 32, bs_step, tuple(jnp.int32(0) for _ in range(6)))
    vs = []
    for idx in range(6):
        ks = lax.bitwise_xor(Ps[idx], jnp.int32(_SGN))
        bb = jnp.where(ks >= 0, ks, lax.bitwise_xor(ks, jnp.int32(0x7FFFFFFF)))
        vs.append(lax.bitcast_convert_type(bb, jnp.float32))
    m = [vs[0] * (1.0 - _FRACS[0]) + vs[1] * _FRACS[0],
         vs[2] * (1.0 - _FRACS[1]) + vs[3] * _FRACS[1],
         vs[4] * (1.0 - _FRACS[2]) + vs[5] * _FRACS[2]]

    def le_row(tl):
        o = obs_s[pl.ds(tl, 1), :]
        return [-0.5 * ((o - m[j]) * (o - m[j]) * inv_cv + lcv) for j in range(3)]

    def valid_row(tl):
        return (lane * _TCH + tl) < _N

    def stepA(tl, M):
        le = le_row(tl)
        valid = valid_row(tl)
        is_first = jnp.logical_and(lane == 0, tl == 0)
        S = []
        for i in range(3):
            for j in range(3):
                if i == j:
                    v = jnp.where(is_first, le[j], _LT[i][j] + le[j])
                    v = jnp.where(valid, v, fzero)
                else:
                    v = jnp.where(is_first, fbig, _LT[i][j] + le[j])
                    v = jnp.where(valid, v, fbig)
                S.append(v)
        out = []
        for i in range(3):
            for j in range(3):
                out.append(jnp.maximum(
                    jnp.maximum(M[3 * i + 0] + S[0 * 3 + j],
                                M[3 * i + 1] + S[1 * 3 + j]),
                    M[3 * i + 2] + S[2 * 3 + j]))
        return tuple(out)

    Minit = tuple(fzero if i == j else fbig for i in range(3) for j in range(3))
    M = lax.fori_loop(0, _TCH, stepA, Minit)

    X = list(M)
    for k in (1, 2, 4, 8, 16, 32, 64):
        sh = []
        for i in range(3):
            for j in range(3):
                r = pltpu.roll(X[3 * i + j], k, axis=1)
                idv = fzero if i == j else fbig
                sh.append(jnp.where(lane < k, idv, r))
        newX = []
        for i in range(3):
            for j in range(3):
                newX.append(jnp.maximum(
                    jnp.maximum(sh[3 * i + 0] + X[0 * 3 + j],
                                sh[3 * i + 1] + X[1 * 3 + j]),
                    sh[3 * i + 2] + X[2 * 3 + j]))
        X = newX
    Pfx = []
    for i in range(3):
        for j in range(3):
            r = pltpu.roll(X[3 * i + j], 1, axis=1)
            idv = fzero if i == j else fbig
            Pfx.append(jnp.where(lane < 1, idv, r))
    vstart = []
    for j in range(3):
        vstart.append(jnp.maximum(
            jnp.maximum(_LS[0] + Pfx[0 * 3 + j], _LS[1] + Pfx[1 * 3 + j]),
            _LS[2] + Pfx[2 * 3 + j]))

    izero = jnp.zeros((1, _CH), jnp.int32)
    ione = jnp.full((1, _CH), 1, jnp.int32)
    itwo = jnp.full((1, _CH), 2, jnp.int32)
    iconst = [izero, ione, itwo]

    def stepC(tl, v):
        le = le_row(tl)
        valid = valid_row(tl)
        is_first = jnp.logical_and(lane == 0, tl == 0)
        newv = []
        for j in range(3):
            c0 = v[0] + _LT[0][j]
            c1 = v[1] + _LT[1][j]
            c2 = v[2] + _LT[2][j]
            b = c0
            p = izero
            u1 = c1 > b
            b = jnp.where(u1, c1, b)
            p = jnp.where(u1, ione, p)
            u2 = c2 > b
            b = jnp.where(u2, c2, b)
            p = jnp.where(u2, itwo, p)
            nv = jnp.where(is_first, v[j] + le[j], b + le[j])
            nv = jnp.where(valid, nv, v[j])
            newv.append(nv)
            pstore = jnp.where(jnp.logical_and(valid, jnp.logical_not(is_first)),
                               p, iconst[j])
            pref = (p0, p1, p2)[j]
            pref[pl.ds(tl, 1), :] = pstore
        return tuple(newv)

    vend = lax.fori_loop(0, _TCH, stepC, tuple(vstart))

    lb = vend[0]
    lp = izero
    u1 = vend[1] > lb
    lb = jnp.where(u1, vend[1], lb)
    lp = jnp.where(u1, ione, lp)
    u2 = vend[2] > lb
    lp = jnp.where(u2, itwo, lp)
    last = jnp.sum(jnp.where(lane == _CH - 1, lp, izero))

    def sel_map(r0, r1, r2, s):
        return jnp.where(s == 0, r0, jnp.where(s == 1, r1, r2))

    pr0 = []
    for j in range(3):
        pref = (p0, p1, p2)[j]
        r = pltpu.roll(pref[pl.ds(0, 1), :], _CH - 1, axis=1)
        pr0.append(jnp.where(lane == _CH - 1, iconst[j], r))

    cur = [izero, ione, itwo]
    nxt = [sel_map(pr0[0], pr0[1], pr0[2], cur[s]) for s in range(3)]
    for s in range(3):
        (w0, w1, w2)[s][pl.ds(_TCH - 1, 1), :] = nxt[s]
    cur = nxt

    def stepD(i, cur):
        tl = _TCH - 2 - i
        r0 = p0[pl.ds(tl + 1, 1), :]
        r1 = p1[pl.ds(tl + 1, 1), :]
        r2 = p2[pl.ds(tl + 1, 1), :]
        n0 = sel_map(r0, r1, r2, cur[0])
        n1 = sel_map(r0, r1, r2, cur[1])
        n2 = sel_map(r0, r1, r2, cur[2])
        w0[pl.ds(tl, 1), :] = n0
        w1[pl.ds(tl, 1), :] = n1
        w2[pl.ds(tl, 1), :] = n2
        return (n0, n1, n2)

    G = lax.fori_loop(0, _TCH - 1, stepD, tuple(cur))

    H = list(G)
    for k in (1, 2, 4, 8, 16, 32, 64):
        Y = []
        for s in range(3):
            r = pltpu.roll(H[s], _CH - k, axis=1)
            Y.append(jnp.where(lane >= _CH - k, iconst[s], r))
        H = [sel_map(H[0], H[1], H[2], Y[s]) for s in range(3)]
    E = []
    for s in range(3):
        r = pltpu.roll(H[s], _CH - 1, axis=1)
        E.append(jnp.where(lane == _CH - 1, iconst[s], r))
    inc = sel_map(E[0], E[1], E[2], jnp.full((1, _CH), 1, jnp.int32) * last)

    sts = jnp.where(inc == 0, w0[...], jnp.where(inc == 1, w1[...], w2[...]))
    out_ref[...] = (sts + 1).astype(jnp.float32)


def _viterbi_states(sm):
    sm_pad = jnp.concatenate([sm, jnp.zeros((_NP - _N,), jnp.float32)])
    sm2d = sm_pad.reshape(_CH, _TCH).T
    out = pl.pallas_call(
        _vit_kernel,
        out_shape=jax.ShapeDtypeStruct((_TCH, _CH), jnp.float32),
        in_specs=[pl.BlockSpec(memory_space=pltpu.VMEM)],
        scratch_shapes=[pltpu.VMEM((_TCH, _CH), jnp.int32)] * 6
        + [pltpu.VMEM((_TCH, _CH), jnp.float32)],
    )(sm2d)
    return out.T.reshape(-1)[:_N]


_E = 320000
_NW = 32
_EW = _E // _NW
_NPAD = 10240
_NROWS = _NPAD // 128


def _sc_scatter_body(st_hbm, row_hbm, col_hbm, out_s, out_c,
                     row_v, col_v, st_v, acc_s, acc_c, idx_v, sh_s, sh_c):
    ci = lax.axis_index("c")
    si = lax.axis_index("s")
    wid = si * 2 + ci

    iota16 = lax.iota(jnp.int32, 16)
    for kk in range(_NROWS // 16):
        idx_v[pl.ds(kk * 16, 16)] = iota16 + kk * 16

    zf = jnp.zeros((16,), jnp.float32)

    def zbody(i, carry):
        for kk in range(8):
            acc_s[i, pl.ds(kk * 16, 16)] = zf
            acc_c[i, pl.ds(kk * 16, 16)] = zf
        return carry

    lax.fori_loop(0, _NROWS, zbody, 0)

    eoff = pl.multiple_of(wid * _EW, 8)
    pltpu.sync_copy(row_hbm.at[pl.ds(eoff, _EW)], row_v)
    pltpu.sync_copy(col_hbm.at[pl.ds(eoff, _EW)], col_v)
    pltpu.sync_copy(st_hbm, st_v)

    @pl.when(si == 0)
    def _zero_shared():
        pltpu.sync_copy(acc_s, sh_s)
        pltpu.sync_copy(acc_c, sh_c)

    plsc.subcore_barrier()

    ones = jnp.full((16,), 1.0, jnp.float32)

    def ebody(i, carry):
        rs, cs = [], []
        for u in range(4):
            off = pl.multiple_of(i * 64 + u * 16, 16)
            rs.append(row_v[pl.ds(off, 16)])
            cs.append(col_v[pl.ds(off, 16)])
        vs = [plsc.load_gather(st_v, [r]) for r in rs]
        for u in range(4):
            rr = lax.shift_right_logical(cs[u], 7)
            ll = lax.bitwise_and(cs[u], 127)
            plsc.addupdate_scatter(acc_s, [rr, ll], vs[u])
            plsc.addupdate_scatter(acc_c, [rr, ll], ones)
        return carry

    lax.fori_loop(0, _EW // 64, ebody, 0)

    pltpu.sync_copy(acc_s, sh_s.at[idx_v], add=True)
    pltpu.sync_copy(acc_c, sh_c.at[idx_v], add=True)

    plsc.subcore_barrier()

    @pl.when(si < _NROWS // 8)
    def _copy_out():
        roff = pl.multiple_of(si * 8, 8)
        ooff = pl.multiple_of(ci * _NROWS + si * 8, 8)
        pltpu.sync_copy(sh_s.at[pl.ds(roff, 8)], out_s.at[pl.ds(ooff, 8)])
        pltpu.sync_copy(sh_c.at[pl.ds(roff, 8)], out_c.at[pl.ds(ooff, 8)])


_sc_scatter = functools.partial(
    pl.kernel,
    out_type=(jax.ShapeDtypeStruct((2 * _NROWS, 128), jnp.float32),
              jax.ShapeDtypeStruct((2 * _NROWS, 128), jnp.float32)),
    mesh=plsc.VectorSubcoreMesh(core_axis_name="c", subcore_axis_name="s"),
    compiler_params=pltpu.CompilerParams(needs_layout_passes=False),
    scratch_types=[
        pltpu.VMEM((_EW,), jnp.int32),
        pltpu.VMEM((_EW,), jnp.int32),
        pltpu.VMEM((_NPAD,), jnp.float32),
        pltpu.VMEM((_NROWS, 128), jnp.float32),
        pltpu.VMEM((_NROWS, 128), jnp.float32),
        pltpu.VMEM((_NROWS,), jnp.int32),
        pltpu.VMEM_SHARED((_NROWS, 128), jnp.float32),
        pltpu.VMEM_SHARED((_NROWS, 128), jnp.float32),
    ],
)(_sc_scatter_body)


def _scatter_mean_sc(states_tensor, row, col):
    st_pad = jnp.zeros((_NPAD,), jnp.float32).at[:_N].set(states_tensor)
    part_s, part_c = _sc_scatter(st_pad, row, col)
    sums = (part_s[:_NROWS] + part_s[_NROWS:]).reshape(-1)[:_N]
    cnt = (part_c[:_NROWS] + part_c[_NROWS:]).reshape(-1)[:_N]
    return jnp.where(cnt > 0, sums / jnp.maximum(cnt, 1.0), 0.0)


def _tail_kernel(recon_ref, nx_ref, s_ref, out_ref, reg_ref):
    recon = recon_ref[...]
    s = s_ref[...]

    R = jnp.sum(recon, axis=1, keepdims=True)
    P = jnp.sum(nx_ref[...], axis=1, keepdims=True)
    a = s * P / (s * R + 1e-8)

    row_min = jnp.min(recon, axis=1, keepdims=True)
    row_max = jnp.max(recon, axis=1, keepdims=True)
    mn = jnp.min(jnp.minimum(a * row_min, a * row_max))
    mx = jnp.max(jnp.maximum(a * row_min, a * row_max))
    mean_nc = jnp.sum(a * R) / (_N * _D)

    rmin = mn * 0.8
    rmax = mx * 1.2
    c1 = (rmax - rmin) / (mx - mn + 1e-8)
    off = rmin - mn * c1
    m2 = mean_nc * c1 + off

    out_ref[...] = recon * (a * (c1 / m2)) + off / m2
    reg_ref[...] = (jnp.sum(recon * recon) * 1e-4).reshape(1, 1)


def kernel(norm_x, reconstructed_features, edge_index):
    recon = reconstructed_features
    spot_mean = jnp.mean(recon, axis=1)

    states_tensor = _viterbi_states(spot_mean)

    row, col = edge_index[0], edge_index[1]
    neighbor_avg = _scatter_mean_sc(states_tensor, row, col)

    smoothed = 0.5 * states_tensor + 0.5 * neighbor_avg

    out, reg = pl.pallas_call(
        _tail_kernel,
        out_shape=(jax.ShapeDtypeStruct((_N, _D), jnp.float32),
                   jax.ShapeDtypeStruct((1, 1), jnp.float32)),
    )(recon, norm_x, smoothed[:, None])

    return out, reg[0, 0]


# speedup vs baseline: 747.4767x; 1.1411x over previous
import functools
import math

import jax
import jax.numpy as jnp
from jax import lax
from jax.experimental import pallas as pl
from jax.experimental.pallas import tpu as pltpu
from jax.experimental.pallas import tpu_sc as plsc

_N = 10000
_D = 128
_S = 3

_TCH = 79
_CH = 128
_NP = _TCH * _CH

_T = 0.01
_LT = [[math.log(1.0 - 2 * _T) if i == j else math.log(_T) for j in range(3)]
       for i in range(3)]
_LS = [math.log(0.1), math.log(0.8), math.log(0.1)]
_BIG = -1e30


_RANKS = (1999.0, 2000.0, 4999.0, 5000.0, 7999.0, 8000.0)
_FRACS = (0.2 * 9999.0 - 1999.0, 0.5 * 9999.0 - 4999.0, 0.8 * 9999.0 - 7999.0)
_SGN = -2147483648


def _vit_kernel(sm_ref, out_ref, p0, p1, p2, w0, w1, w2, obs_s):
    lane = lax.broadcasted_iota(jnp.int32, (1, _CH), 1)
    fbig = jnp.full((1, _CH), _BIG, jnp.float32)
    fzero = jnp.zeros((1, _CH), jnp.float32)

    t_iota = lax.broadcasted_iota(jnp.int32, (_TCH, _CH), 0)
    c_iota = lax.broadcasted_iota(jnp.int32, (_TCH, _CH), 1)
    valid2d = (c_iota * _TCH + t_iota) < _N
    sm = sm_ref[...]
    mean = jnp.sum(jnp.where(valid2d, sm, 0.0)) / _N
    d = sm - mean
    varsm = jnp.sum(jnp.where(valid2d, d * d, 0.0)) / _N
    stde = jnp.sqrt(varsm) + 1e-8
    obs = jnp.where(valid2d, d / stde, 0.0)
    obs_s[...] = obs
    cv = varsm / (stde * stde) + 1e-4
    inv_cv = 1.0 / cv
    lcv = jnp.log(2.0 * jnp.pi * cv)

    b = lax.bitcast_convert_type(obs, jnp.int32)
    key = lax.bitwise_xor(
        b, lax.bitwise_and(lax.shift_right_arithmetic(b, 31),
                           jnp.int32(0x7FFFFFFF)))
    key = jnp.where(valid2d, key, jnp.int32(0x7FFFFFFF))

    def bs_step(i, Ps):
        bitv = lax.shift_left(jnp.int32(1), 31 - i)
        out = []
        for idx in range(6):
            C = lax.bitwise_or(Ps[idx], bitv)
            th = lax.bitwise_xor(C, jnp.int32(_SGN))
            cnt = jnp.sum(jnp.where(key < th, 1.0, 0.0))
            out.append(jnp.where(cnt <= _RANKS[idx], C, Ps[idx]))
        return tuple(out)

    Ps = lax.fori_loop(0, 32, bs_step, tuple(jnp.int32(0) for _ in range(6)))
    vs = []
    for idx in range(6):
        ks = lax.bitwise_xor(Ps[idx], jnp.int32(_SGN))
        bb = jnp.where(ks >= 0, ks, lax.bitwise_xor(ks, jnp.int32(0x7FFFFFFF)))
        vs.append(lax.bitcast_convert_type(bb, jnp.float32))
    m = [vs[0] * (1.0 - _FRACS[0]) + vs[1] * _FRACS[0],
         vs[2] * (1.0 - _FRACS[1]) + vs[3] * _FRACS[1],
         vs[4] * (1.0 - _FRACS[2]) + vs[5] * _FRACS[2]]

    def le_row(tl):
        o = obs_s[pl.ds(tl, 1), :]
        return [-0.5 * ((o - m[j]) * (o - m[j]) * inv_cv + lcv) for j in range(3)]

    def valid_row(tl):
        return (lane * _TCH + tl) < _N

    def stepA(tl, M):
        le = le_row(tl)
        valid = valid_row(tl)
        is_first = jnp.logical_and(lane == 0, tl == 0)
        S = []
        for i in range(3):
            for j in range(3):
                if i == j:
                    v = jnp.where(is_first, le[j], _LT[i][j] + le[j])
                    v = jnp.where(valid, v, fzero)
                else:
                    v = jnp.where(is_first, fbig, _LT[i][j] + le[j])
                    v = jnp.where(valid, v, fbig)
                S.append(v)
        out = []
        for i in range(3):
            for j in range(3):
                out.append(jnp.maximum(
                    jnp.maximum(M[3 * i + 0] + S[0 * 3 + j],
                                M[3 * i + 1] + S[1 * 3 + j]),
                    M[3 * i + 2] + S[2 * 3 + j]))
        return tuple(out)

    Minit = tuple(fzero if i == j else fbig for i in range(3) for j in range(3))
    M = lax.fori_loop(0, _TCH, stepA, Minit)

    X = list(M)
    for k in (1, 2, 4, 8, 16, 32, 64):
        sh = []
        for i in range(3):
            for j in range(3):
                r = pltpu.roll(X[3 * i + j], k, axis=1)
                idv = fzero if i == j else fbig
                sh.append(jnp.where(lane < k, idv, r))
        newX = []
        for i in range(3):
            for j in range(3):
                newX.append(jnp.maximum(
                    jnp.maximum(sh[3 * i + 0] + X[0 * 3 + j],
                                sh[3 * i + 1] + X[1 * 3 + j]),
                    sh[3 * i + 2] + X[2 * 3 + j]))
        X = newX
    Pfx = []
    for i in range(3):
        for j in range(3):
            r = pltpu.roll(X[3 * i + j], 1, axis=1)
            idv = fzero if i == j else fbig
            Pfx.append(jnp.where(lane < 1, idv, r))
    vstart = []
    for j in range(3):
        vstart.append(jnp.maximum(
            jnp.maximum(_LS[0] + Pfx[0 * 3 + j], _LS[1] + Pfx[1 * 3 + j]),
            _LS[2] + Pfx[2 * 3 + j]))

    izero = jnp.zeros((1, _CH), jnp.int32)
    ione = jnp.full((1, _CH), 1, jnp.int32)
    itwo = jnp.full((1, _CH), 2, jnp.int32)
    iconst = [izero, ione, itwo]

    def stepC(tl, v):
        le = le_row(tl)
        valid = valid_row(tl)
        is_first = jnp.logical_and(lane == 0, tl == 0)
        newv = []
        for j in range(3):
            c0 = v[0] + _LT[0][j]
            c1 = v[1] + _LT[1][j]
            c2 = v[2] + _LT[2][j]
            b = c0
            p = izero
            u1 = c1 > b
            b = jnp.where(u1, c1, b)
            p = jnp.where(u1, ione, p)
            u2 = c2 > b
            b = jnp.where(u2, c2, b)
            p = jnp.where(u2, itwo, p)
            nv = jnp.where(is_first, v[j] + le[j], b + le[j])
            nv = jnp.where(valid, nv, v[j])
            newv.append(nv)
            pstore = jnp.where(jnp.logical_and(valid, jnp.logical_not(is_first)),
                               p, iconst[j])
            pref = (p0, p1, p2)[j]
            pref[pl.ds(tl, 1), :] = pstore
        return tuple(newv)

    vend = lax.fori_loop(0, _TCH, stepC, tuple(vstart))

    lb = vend[0]
    lp = izero
    u1 = vend[1] > lb
    lb = jnp.where(u1, vend[1], lb)
    lp = jnp.where(u1, ione, lp)
    u2 = vend[2] > lb
    lp = jnp.where(u2, itwo, lp)
    last = jnp.sum(jnp.where(lane == _CH - 1, lp, izero))

    def sel_map(r0, r1, r2, s):
        return jnp.where(s == 0, r0, jnp.where(s == 1, r1, r2))

    pr0 = []
    for j in range(3):
        pref = (p0, p1, p2)[j]
        r = pltpu.roll(pref[pl.ds(0, 1), :], _CH - 1, axis=1)
        pr0.append(jnp.where(lane == _CH - 1, iconst[j], r))

    cur = [izero, ione, itwo]
    nxt = [sel_map(pr0[0], pr0[1], pr0[2], cur[s]) for s in range(3)]
    for s in range(3):
        (w0, w1, w2)[s][pl.ds(_TCH - 1, 1), :] = nxt[s]
    cur = nxt

    def stepD(i, cur):
        tl = _TCH - 2 - i
        r0 = p0[pl.ds(tl + 1, 1), :]
        r1 = p1[pl.ds(tl + 1, 1), :]
        r2 = p2[pl.ds(tl + 1, 1), :]
        n0 = sel_map(r0, r1, r2, cur[0])
        n1 = sel_map(r0, r1, r2, cur[1])
        n2 = sel_map(r0, r1, r2, cur[2])
        w0[pl.ds(tl, 1), :] = n0
        w1[pl.ds(tl, 1), :] = n1
        w2[pl.ds(tl, 1), :] = n2
        return (n0, n1, n2)

    G = lax.fori_loop(0, _TCH - 1, stepD, tuple(cur))

    H = list(G)
    for k in (1, 2, 4, 8, 16, 32, 64):
        Y = []
        for s in range(3):
            r = pltpu.roll(H[s], _CH - k, axis=1)
            Y.append(jnp.where(lane >= _CH - k, iconst[s], r))
        H = [sel_map(H[0], H[1], H[2], Y[s]) for s in range(3)]
    E = []
    for s in range(3):
        r = pltpu.roll(H[s], _CH - 1, axis=1)
        E.append(jnp.where(lane == _CH - 1, iconst[s], r))
    inc = sel_map(E[0], E[1], E[2], jnp.full((1, _CH), 1, jnp.int32) * last)

    sts = jnp.where(inc == 0, w0[...], jnp.where(inc == 1, w1[...], w2[...]))
    out_ref[...] = (sts + 1).astype(jnp.float32)


def _viterbi_states(sm):
    sm_pad = jnp.concatenate([sm, jnp.zeros((_NP - _N,), jnp.float32)])
    sm2d = sm_pad.reshape(_CH, _TCH).T
    out = pl.pallas_call(
        _vit_kernel,
        out_shape=jax.ShapeDtypeStruct((_TCH, _CH), jnp.float32),
        in_specs=[pl.BlockSpec(memory_space=pltpu.VMEM)],
        scratch_shapes=[pltpu.VMEM((_TCH, _CH), jnp.int32)] * 6
        + [pltpu.VMEM((_TCH, _CH), jnp.float32)],
    )(sm2d)
    return out


_E = 320000
_NW = 32
_EW = _E // _NW
_NPAD = 10240
_NROWS = _NPAD // 128


def _sc_scatter_body(st_hbm, edge_hbm, out_s, out_c,
                     row_v, col_v, st_v, acc_s, acc_c, idx_v, sh_s, sh_c):
    ci = lax.axis_index("c")
    si = lax.axis_index("s")
    wid = si * 2 + ci

    iota16 = lax.iota(jnp.int32, 16)
    for kk in range(_NROWS // 16):
        idx_v[pl.ds(kk * 16, 16)] = iota16 + kk * 16

    zf = jnp.zeros((16,), jnp.float32)

    def zbody(i, carry):
        for kk in range(8):
            acc_s[i, pl.ds(kk * 16, 16)] = zf
            acc_c[i, pl.ds(kk * 16, 16)] = zf
        return carry

    lax.fori_loop(0, _NROWS, zbody, 0)

    eoff = pl.multiple_of(wid * _EW, 8)
    coff = pl.multiple_of(_E + wid * _EW, 8)
    pltpu.sync_copy(edge_hbm.at[pl.ds(eoff, _EW)], row_v)
    pltpu.sync_copy(edge_hbm.at[pl.ds(coff, _EW)], col_v)
    pltpu.sync_copy(st_hbm, st_v)

    @pl.when(si == 0)
    def _zero_shared():
        pltpu.sync_copy(acc_s, sh_s)
        pltpu.sync_copy(acc_c, sh_c)

    plsc.subcore_barrier()

    ones = jnp.full((16,), 1.0, jnp.float32)

    def ebody(i, carry):
        rs, cs = [], []
        for u in range(4):
            off = pl.multiple_of(i * 64 + u * 16, 16)
            rs.append(row_v[pl.ds(off, 16)])
            cs.append(col_v[pl.ds(off, 16)])
        def remap(r):
            q = lax.shift_right_logical(r * 53093, 22)
            return lax.shift_left(r, 7) - 10111 * q

        vs = [plsc.load_gather(st_v, [remap(r)]) for r in rs]
        for u in range(4):
            rr = lax.shift_right_logical(cs[u], 7)
            ll = lax.bitwise_and(cs[u], 127)
            plsc.addupdate_scatter(acc_s, [rr, ll], vs[u])
            plsc.addupdate_scatter(acc_c, [rr, ll], ones)
        return carry

    lax.fori_loop(0, _EW // 64, ebody, 0)

    pltpu.sync_copy(acc_s, sh_s.at[idx_v], add=True)
    pltpu.sync_copy(acc_c, sh_c.at[idx_v], add=True)

    plsc.subcore_barrier()

    @pl.when(si < _NROWS // 8)
    def _copy_out():
        roff = pl.multiple_of(si * 8, 8)
        ooff = pl.multiple_of(ci * _NROWS + si * 8, 8)
        pltpu.sync_copy(sh_s.at[pl.ds(roff, 8)], out_s.at[pl.ds(ooff, 8)])
        pltpu.sync_copy(sh_c.at[pl.ds(roff, 8)], out_c.at[pl.ds(ooff, 8)])


_sc_scatter = functools.partial(
    pl.kernel,
    out_type=(jax.ShapeDtypeStruct((2 * _NROWS, 128), jnp.float32),
              jax.ShapeDtypeStruct((2 * _NROWS, 128), jnp.float32)),
    mesh=plsc.VectorSubcoreMesh(core_axis_name="c", subcore_axis_name="s"),
    compiler_params=pltpu.CompilerParams(needs_layout_passes=False),
    scratch_types=[
        pltpu.VMEM((_EW,), jnp.int32),
        pltpu.VMEM((_EW,), jnp.int32),
        pltpu.VMEM((_NP,), jnp.float32),
        pltpu.VMEM((_NROWS, 128), jnp.float32),
        pltpu.VMEM((_NROWS, 128), jnp.float32),
        pltpu.VMEM((_NROWS,), jnp.int32),
        pltpu.VMEM_SHARED((_NROWS, 128), jnp.float32),
        pltpu.VMEM_SHARED((_NROWS, 128), jnp.float32),
    ],
)(_sc_scatter_body)


def _scatter_mean_sc(st2d, edge_index):
    part_s, part_c = _sc_scatter(st2d.reshape(-1), edge_index.reshape(-1))
    sums = (part_s[:_NROWS] + part_s[_NROWS:]).reshape(-1)[:_N]
    cnt = (part_c[:_NROWS] + part_c[_NROWS:]).reshape(-1)[:_N]
    return jnp.where(cnt > 0, sums / jnp.maximum(cnt, 1.0), 0.0)


def _tail_kernel(recon_ref, nx_ref, s_ref, out_ref, reg_ref):
    recon = recon_ref[...]
    s = s_ref[...]

    R = jnp.sum(recon, axis=1, keepdims=True)
    P = jnp.sum(nx_ref[...], axis=1, keepdims=True)
    a = s * P / (s * R + 1e-8)

    row_min = jnp.min(recon, axis=1, keepdims=True)
    row_max = jnp.max(recon, axis=1, keepdims=True)
    mn = jnp.min(jnp.minimum(a * row_min, a * row_max))
    mx = jnp.max(jnp.maximum(a * row_min, a * row_max))
    mean_nc = jnp.sum(a * R) / (_N * _D)

    rmin = mn * 0.8
    rmax = mx * 1.2
    c1 = (rmax - rmin) / (mx - mn + 1e-8)
    off = rmin - mn * c1
    m2 = mean_nc * c1 + off

    out_ref[...] = recon * (a * (c1 / m2)) + off / m2
    reg_ref[...] = (jnp.sum(recon * recon) * 1e-4).reshape(1, 1)


def kernel(norm_x, reconstructed_features, edge_index):
    recon = reconstructed_features
    spot_mean = jnp.mean(recon, axis=1)

    st2d = _viterbi_states(spot_mean)
    states_tensor = st2d.T.reshape(-1)[:_N]

    neighbor_avg = _scatter_mean_sc(st2d, edge_index)

    smoothed = 0.5 * states_tensor + 0.5 * neighbor_avg

    out, reg = pl.pallas_call(
        _tail_kernel,
        out_shape=(jax.ShapeDtypeStruct((_N, _D), jnp.float32),
                   jax.ShapeDtypeStruct((1, 1), jnp.float32)),
    )(recon, norm_x, smoothed[:, None])

    return out, reg[0, 0]
